# Initial kernel scaffold; baseline (speedup 1.0000x reference)
import functools

import jax
import jax.numpy as jnp
from jax import lax
from jax.experimental import pallas as pl
from jax.experimental.pallas import tpu as pltpu
from jax.experimental.pallas import tpu_sc as plsc

N = 10000
E = 320000
D = 128
H = 128
G = 128
OUT = 2

NC, NS = 2, 16
NW = NC * NS
CH = 128
TOTCH = E // CH
NPS = N // NS
RB = 1000
NB = N // RB

_NBCH = N // CH
_NTAIL = N - _NBCH * CH

_f32 = jnp.float32
_i32 = jnp.int32


def _stats_body(dst_hbm, batch_hbm, x_hbm, deg_out, newsx_out,
                deg_sp, bat_sp, idx_v, tidx_v, ones_v, bat_v, fidx_v,
                news_v, zrow_v, sem):
    cid = lax.axis_index("c")
    sid = lax.axis_index("s")
    wid = sid * NC + cid

    for k in range(CH // 16):
        ones_v[pl.ds(16 * k, 16)] = jnp.full((16,), 1.0, _f32)
    for k in range(G // 16):
        zrow_v[pl.ds(16 * k, 16)] = jnp.zeros((16,), _f32)

    @pl.when(sid == 0)
    def _():
        def zb(i, c):
            pltpu.sync_copy(zrow_v, deg_sp.at[pl.ds(i * G, G)])
            return c
        lax.fori_loop(0, N // G, zb, 0)
        tail = N - (N // G) * G
        if tail:
            pltpu.sync_copy(zrow_v.at[pl.ds(0, tail)],
                            deg_sp.at[pl.ds((N // G) * G, tail)])
        pltpu.sync_copy(zrow_v, bat_sp)

    plsc.subcore_barrier()

    n_i = (TOTCH - wid + NW - 1) // NW

    def deg_step(i, c):
        base = (wid + i * NW) * CH
        pltpu.sync_copy(dst_hbm.at[pl.ds(base, CH)], idx_v)
        pltpu.sync_copy(ones_v, deg_sp.at[idx_v], add=True)
        return c

    lax.fori_loop(0, n_i, deg_step, 0)

    @pl.when(jnp.logical_and(cid == 0, sid == 0))
    def _():
        def bat_step(i, c):
            pltpu.sync_copy(batch_hbm.at[pl.ds(i * CH, CH)], idx_v)
            pltpu.sync_copy(ones_v, bat_sp.at[idx_v], add=True)
            return c
        lax.fori_loop(0, _NBCH, bat_step, 0)
        if _NTAIL:
            pltpu.sync_copy(batch_hbm.at[pl.ds(_NBCH * CH, _NTAIL)], tidx_v)
            pltpu.sync_copy(ones_v.at[pl.ds(0, _NTAIL)],
                            bat_sp.at[tidx_v], add=True)

    plsc.subcore_barrier()

    @pl.when(sid == 0)
    def _():
        pltpu.sync_copy(deg_sp, deg_out.at[cid])

    @pl.when(jnp.logical_and(cid == 0, sid == 0))
    def _():
        pltpu.sync_copy(bat_sp, bat_v)
        carry = jnp.float32(0.0)
        for k in range(G // 16):
            v = bat_v[pl.ds(16 * k, 16)]
            s = plsc.cumsum(v)
            excl = s - v + carry
            fidx_v[pl.ds(16 * k, 16)] = jnp.minimum(
                excl, float(N - 1)).astype(_i32)
            carry = carry + jnp.sum(v)
        pltpu.async_copy(x_hbm.at[fidx_v], news_v, sem).wait()
        pltpu.sync_copy(news_v, newsx_out)


def _stats_call(dst, batch, x):
    mesh = plsc.VectorSubcoreMesh(core_axis_name="c", subcore_axis_name="s")
    return pl.kernel(
        _stats_body,
        out_type=(jax.ShapeDtypeStruct((NC, N), _f32),
                  jax.ShapeDtypeStruct((G, D), _f32)),
        mesh=mesh,
        scratch_types=[
            pltpu.VMEM_SHARED((N,), _f32),
            pltpu.VMEM_SHARED((G,), _f32),
            pltpu.VMEM((CH,), _i32),
            pltpu.VMEM((_NTAIL or 8,), _i32),
            pltpu.VMEM((CH,), _f32),
            pltpu.VMEM((G,), _f32),
            pltpu.VMEM((G,), _i32),
            pltpu.VMEM((G, D), _f32),
            pltpu.VMEM((G,), _f32),
            pltpu.SemaphoreType.DMA,
        ],
    )(dst, batch, x)


def _scatter_body(src_hbm, dst_hbm, y_hbm, z_hbm, aggp_out,
                  acc_sp, sidx_v, didx_v, rows_v, sem):
    cid = lax.axis_index("c")
    sid = lax.axis_index("s")
    wid = sid * NC + cid

    pltpu.sync_copy(z_hbm.at[pl.ds(sid * NPS, NPS)],
                    acc_sp.at[pl.ds(sid * NPS, NPS)])
    plsc.subcore_barrier()

    n_i = (TOTCH - wid + NW - 1) // NW

    def step(i, c):
        base = (wid + i * NW) * CH
        pltpu.sync_copy(src_hbm.at[pl.ds(base, CH)], sidx_v)
        pltpu.sync_copy(dst_hbm.at[pl.ds(base, CH)], didx_v)
        pltpu.async_copy(y_hbm.at[sidx_v], rows_v, sem).wait()
        pltpu.sync_copy(rows_v, acc_sp.at[didx_v], add=True)
        return c

    lax.fori_loop(0, n_i, step, 0)
    plsc.subcore_barrier()

    pltpu.sync_copy(acc_sp.at[pl.ds(sid * NPS, NPS)],
                    aggp_out.at[cid, pl.ds(sid * NPS, NPS)])


def _scatter_call(src, dst, y, z):
    mesh = plsc.VectorSubcoreMesh(core_axis_name="c", subcore_axis_name="s")
    return pl.kernel(
        _scatter_body,
        out_type=jax.ShapeDtypeStruct((NC, N, H), _f32),
        mesh=mesh,
        scratch_types=[
            pltpu.VMEM_SHARED((N, H), _f32),
            pltpu.VMEM((CH,), _i32),
            pltpu.VMEM((CH,), _i32),
            pltpu.VMEM((CH, H), _f32),
            pltpu.SemaphoreType.DMA,
        ],
    )(src, dst, y, z)


def _dense_body(x_ref, degT_ref, W_ref, newsx_ref, lin0W_ref, lin0b_ref,
                y_ref, dinv_ref, news_ref):
    pid = pl.program_id(0)
    deg = degT_ref[:, 0:1] + degT_ref[:, 1:2] + 1.0
    dinv = lax.rsqrt(deg)
    xw = jnp.dot(x_ref[...], W_ref[...], preferred_element_type=_f32)
    y_ref[...] = xw * dinv
    dinv_ref[...] = dinv

    @pl.when(pid == 0)
    def _():
        nw = jnp.dot(newsx_ref[...], lin0W_ref[...],
                     preferred_element_type=_f32)
        news_ref[...] = jnp.maximum(nw + lin0b_ref[...], 0.0)


def _dense_call(x, degT, W, newsx, lin0_W, lin0_b):
    return pl.pallas_call(
        _dense_body,
        grid=(NB,),
        in_specs=[
            pl.BlockSpec((RB, D), lambda i: (i, 0)),
            pl.BlockSpec((RB, NC), lambda i: (i, 0)),
            pl.BlockSpec((D, H), lambda i: (0, 0)),
            pl.BlockSpec((G, D), lambda i: (0, 0)),
            pl.BlockSpec((D, H), lambda i: (0, 0)),
            pl.BlockSpec((1, H), lambda i: (0, 0)),
        ],
        out_specs=[
            pl.BlockSpec((RB, H), lambda i: (i, 0)),
            pl.BlockSpec((RB, 1), lambda i: (i, 0)),
            pl.BlockSpec((G, H), lambda i: (0, 0)),
        ],
        out_shape=[
            jax.ShapeDtypeStruct((N, H), _f32),
            jax.ShapeDtypeStruct((N, 1), _f32),
            jax.ShapeDtypeStruct((G, H), _f32),
        ],
    )(x, degT, W, newsx, lin0_W, lin0_b)


def _readout_body(y_ref, dinv_ref, aggp_ref, b_ref, bat_ref, news_ref,
                  lin1W_ref, lin1b_ref, lin2W_ref, lin2b_ref,
                  out_ref, pooled_ref):
    pid = pl.program_id(0)

    @pl.when(pid == 0)
    def _():
        pooled_ref[...] = jnp.full((G, H), -jnp.inf, _f32)

    agg = aggp_ref[0] + aggp_ref[1] + y_ref[...]
    conv = jnp.maximum(agg * dinv_ref[...] + b_ref[...], 0.0)

    bat = bat_ref[...]
    bmin = jnp.min(bat)
    bmax = jnp.max(bat)

    def pool_step(g, c):
        m = jnp.max(jnp.where(bat == g, conv, -jnp.inf), axis=0,
                    keepdims=True)
        cur = pooled_ref[pl.ds(g, 1), :]
        pooled_ref[pl.ds(g, 1), :] = jnp.maximum(cur, m)
        return c

    lax.fori_loop(bmin, bmax + 1, pool_step, 0)

    @pl.when(pid == NB - 1)
    def _():
        pooled = pooled_ref[...]
        h1 = (jnp.dot(pooled, lin1W_ref[0:H, :], preferred_element_type=_f32)
              + jnp.dot(news_ref[...], lin1W_ref[H:2 * H, :],
                        preferred_element_type=_f32)
              + lin1b_ref[...])
        h1 = jnp.maximum(h1, 0.0)
        h2 = jnp.dot(h1, lin2W_ref[...], preferred_element_type=_f32) \
            + lin2b_ref[...]
        m = jnp.max(h2, axis=-1, keepdims=True)
        lse = m + jnp.log(jnp.sum(jnp.exp(h2 - m), axis=-1, keepdims=True))
        out_ref[...] = h2 - lse


def _readout_call(y, dinv, aggp, b, bat2, news, lin1_W, lin1_b, lin2_W,
                  lin2_b):
    return pl.pallas_call(
        _readout_body,
        grid=(NB,),
        in_specs=[
            pl.BlockSpec((RB, H), lambda i: (i, 0)),
            pl.BlockSpec((RB, 1), lambda i: (i, 0)),
            pl.BlockSpec((NC, RB, H), lambda i: (0, i, 0)),
            pl.BlockSpec((1, H), lambda i: (0, 0)),
            pl.BlockSpec((RB, 1), lambda i: (i, 0)),
            pl.BlockSpec((G, H), lambda i: (0, 0)),
            pl.BlockSpec((2 * H, H), lambda i: (0, 0)),
            pl.BlockSpec((1, H), lambda i: (0, 0)),
            pl.BlockSpec((H, OUT), lambda i: (0, 0)),
            pl.BlockSpec((1, OUT), lambda i: (0, 0)),
        ],
        out_specs=pl.BlockSpec((G, OUT), lambda i: (0, 0)),
        out_shape=jax.ShapeDtypeStruct((G, OUT), _f32),
        scratch_shapes=[pltpu.VMEM((G, H), _f32)],
    )(y, dinv, aggp, b, bat2, news, lin1_W, lin1_b, lin2_W, lin2_b)


@jax.jit
def kernel(x, edge_index, batch, W, b, lin0_W, lin0_b, lin1_W, lin1_b,
           lin2_W, lin2_b):
    src = edge_index[0]
    dst = edge_index[1]
    batch = batch.astype(_i32)

    degp, newsx = _stats_call(dst, batch, x)
    degT = degp.T

    y, dinv, news = _dense_call(x, degT, W, newsx, lin0_W,
                                lin0_b.reshape(1, H))

    z = jnp.zeros((N, H), _f32)
    aggp = _scatter_call(src, dst, y, z)

    out = _readout_call(y, dinv, aggp, b.reshape(1, H),
                        batch.reshape(N, 1), news, lin1_W,
                        lin1_b.reshape(1, H), lin2_W,
                        lin2_b.reshape(1, OUT))
    return out


# speedup vs baseline: 20.1170x; 20.1170x over previous
import functools

import jax
import jax.numpy as jnp
from jax import lax
from jax.experimental import pallas as pl
from jax.experimental.pallas import tpu as pltpu
from jax.experimental.pallas import tpu_sc as plsc

N = 10000
E = 320000
D = 128
H = 128
G = 128
OUT = 2

NC, NS = 2, 16
NW = NC * NS
CH = 128
TOTCH = E // CH
NPS8 = 632
LASTN = N - NPS8 * (NS - 1)
RB = 1000
NB = N // RB

_NBCH = N // CH
_NTAIL = N - _NBCH * CH

_f32 = jnp.float32
_i32 = jnp.int32


def _stats_body(dst_hbm, deg_out, deg_sp, idx_v, ones_v, zrow_v):
    cid = lax.axis_index("c")
    sid = lax.axis_index("s")
    wid = sid * NC + cid

    for k in range(CH // 16):
        ones_v[pl.ds(16 * k, 16)] = jnp.full((16,), 1.0, _f32)
    for k in range(G // 16):
        zrow_v[pl.ds(16 * k, 16)] = jnp.zeros((16,), _f32)

    @pl.when(sid == 0)
    def _():
        def zb(i, c):
            pltpu.sync_copy(zrow_v, deg_sp.at[pl.ds(i * G, G)])
            return c
        lax.fori_loop(0, N // G, zb, 0)
        tail = N - (N // G) * G
        if tail:
            pltpu.sync_copy(zrow_v.at[pl.ds(0, tail)],
                            deg_sp.at[pl.ds((N // G) * G, tail)])

    plsc.subcore_barrier()

    n_i = (TOTCH - wid + NW - 1) // NW

    def deg_step(i, c):
        base = (wid + i * NW) * CH
        pltpu.sync_copy(dst_hbm.at[pl.ds(base, CH)], idx_v)
        pltpu.sync_copy(ones_v, deg_sp.at[idx_v], add=True)
        return c

    lax.fori_loop(0, n_i, deg_step, 0)

    plsc.subcore_barrier()

    @pl.when(sid == 0)
    def _():
        pltpu.sync_copy(deg_sp, deg_out.at[cid])


def _stats_call(dst):
    mesh = plsc.VectorSubcoreMesh(core_axis_name="c", subcore_axis_name="s")
    return pl.kernel(
        _stats_body,
        out_type=jax.ShapeDtypeStruct((NC, N), _f32),
        mesh=mesh,
        scratch_types=[
            pltpu.VMEM_SHARED((N,), _f32),
            pltpu.VMEM((CH,), _i32),
            pltpu.VMEM((CH,), _f32),
            pltpu.VMEM((G,), _f32),
        ],
    )(dst)


def _scatter_body(src_hbm, dst_hbm, y_hbm, z_hbm, aggp_out,
                  acc_sp, sidx_v, didx_v, rows_v, sem):
    cid = lax.axis_index("c")
    sid = lax.axis_index("s")
    wid = sid * NC + cid

    for k in range(NS):
        sz = NPS8 if k < NS - 1 else LASTN

        @pl.when(sid == k)
        def _(k=k, sz=sz):
            pltpu.sync_copy(z_hbm.at[pl.ds(k * NPS8, sz)],
                            acc_sp.at[pl.ds(k * NPS8, sz)])

    plsc.subcore_barrier()

    n_i = (TOTCH - wid + NW - 1) // NW

    def step(i, c):
        base = (wid + i * NW) * CH
        pltpu.sync_copy(src_hbm.at[pl.ds(base, CH)], sidx_v)
        pltpu.sync_copy(dst_hbm.at[pl.ds(base, CH)], didx_v)
        pltpu.async_copy(y_hbm.at[sidx_v], rows_v, sem).wait()
        pltpu.sync_copy(rows_v, acc_sp.at[didx_v], add=True)
        return c

    lax.fori_loop(0, n_i, step, 0)
    plsc.subcore_barrier()

    for k in range(NS):
        sz = NPS8 if k < NS - 1 else LASTN

        @pl.when(sid == k)
        def _(k=k, sz=sz):
            pltpu.sync_copy(acc_sp.at[pl.ds(k * NPS8, sz)],
                            aggp_out.at[cid, pl.ds(k * NPS8, sz)])


def _scatter_call(src, dst, y, z):
    mesh = plsc.VectorSubcoreMesh(core_axis_name="c", subcore_axis_name="s")
    return pl.kernel(
        _scatter_body,
        out_type=jax.ShapeDtypeStruct((NC, N, H), _f32),
        mesh=mesh,
        scratch_types=[
            pltpu.VMEM_SHARED((N, H), _f32),
            pltpu.VMEM((CH,), _i32),
            pltpu.VMEM((CH,), _i32),
            pltpu.VMEM((CH, H), _f32),
            pltpu.SemaphoreType.DMA,
        ],
    )(src, dst, y, z)


def _dense_body(x_ref, degT_ref, W_ref, batT_ref, xfull_ref, lin0W_ref,
                lin0b_ref, y_ref, dinv_ref, news_ref):
    pid = pl.program_id(0)
    deg = degT_ref[:, 0:1] + degT_ref[:, 1:2] + 1.0
    dinv = lax.rsqrt(deg)
    xw = jnp.dot(x_ref[...], W_ref[...], preferred_element_type=_f32)
    y_ref[...] = xw * dinv
    dinv_ref[...] = dinv

    @pl.when(pid == NB - 1)
    def _():
        bat = batT_ref[...]
        gi = lax.broadcasted_iota(_i32, (G, 1), 0)
        cnt = jnp.sum((bat < gi).astype(_f32), axis=1, keepdims=True)
        fidx = jnp.minimum(cnt, float(N - 1)).astype(_i32)
        ni = lax.broadcasted_iota(_i32, (G, N), 1)
        oh = (ni == fidx).astype(_f32)
        newsx = jnp.dot(oh, xfull_ref[...], preferred_element_type=_f32)
        nw = jnp.dot(newsx, lin0W_ref[...], preferred_element_type=_f32)
        news_ref[...] = jnp.maximum(nw + lin0b_ref[...], 0.0)


def _dense_call(x, degT, W, batT, lin0_W, lin0_b):
    return pl.pallas_call(
        _dense_body,
        grid=(NB,),
        in_specs=[
            pl.BlockSpec((RB, D), lambda i: (i, 0)),
            pl.BlockSpec((RB, NC), lambda i: (i, 0)),
            pl.BlockSpec((D, H), lambda i: (0, 0)),
            pl.BlockSpec((1, N), lambda i: (0, 0)),
            pl.BlockSpec((N, D), lambda i: (0, 0)),
            pl.BlockSpec((D, H), lambda i: (0, 0)),
            pl.BlockSpec((1, H), lambda i: (0, 0)),
        ],
        out_specs=[
            pl.BlockSpec((RB, H), lambda i: (i, 0)),
            pl.BlockSpec((RB, 1), lambda i: (i, 0)),
            pl.BlockSpec((G, H), lambda i: (0, 0)),
        ],
        out_shape=[
            jax.ShapeDtypeStruct((N, H), _f32),
            jax.ShapeDtypeStruct((N, 1), _f32),
            jax.ShapeDtypeStruct((G, H), _f32),
        ],
    )(x, degT, W, batT, x, lin0_W, lin0_b)


def _readout_body(y_ref, dinv_ref, aggp_ref, b_ref, bat_ref, news_ref,
                  lin1W_ref, lin1b_ref, lin2W_ref, lin2b_ref,
                  out_ref, pooled_ref):
    pid = pl.program_id(0)

    @pl.when(pid == 0)
    def _():
        pooled_ref[...] = jnp.full((G, H), -jnp.inf, _f32)

    agg = aggp_ref[0] + aggp_ref[1] + y_ref[...]
    conv = jnp.maximum(agg * dinv_ref[...] + b_ref[...], 0.0)

    bat = bat_ref[...]
    bmin = jnp.min(bat)
    bmax = jnp.max(bat)

    def pool_step(g, c):
        m = jnp.max(jnp.where(bat == g, conv, -jnp.inf), axis=0,
                    keepdims=True)
        cur = pooled_ref[pl.ds(g, 1), :]
        pooled_ref[pl.ds(g, 1), :] = jnp.maximum(cur, m)
        return c

    lax.fori_loop(bmin, bmax + 1, pool_step, 0)

    @pl.when(pid == NB - 1)
    def _():
        pooled = pooled_ref[...]
        h1 = (jnp.dot(pooled, lin1W_ref[0:H, :], preferred_element_type=_f32)
              + jnp.dot(news_ref[...], lin1W_ref[H:2 * H, :],
                        preferred_element_type=_f32)
              + lin1b_ref[...])
        h1 = jnp.maximum(h1, 0.0)
        h2 = jnp.dot(h1, lin2W_ref[...], preferred_element_type=_f32) \
            + lin2b_ref[...]
        m = jnp.max(h2, axis=-1, keepdims=True)
        lse = m + jnp.log(jnp.sum(jnp.exp(h2 - m), axis=-1, keepdims=True))
        out_ref[...] = h2 - lse


def _readout_call(y, dinv, aggp, b, bat2, news, lin1_W, lin1_b, lin2_W,
                  lin2_b):
    return pl.pallas_call(
        _readout_body,
        grid=(NB,),
        in_specs=[
            pl.BlockSpec((RB, H), lambda i: (i, 0)),
            pl.BlockSpec((RB, 1), lambda i: (i, 0)),
            pl.BlockSpec((NC, RB, H), lambda i: (0, i, 0)),
            pl.BlockSpec((1, H), lambda i: (0, 0)),
            pl.BlockSpec((RB, 1), lambda i: (i, 0)),
            pl.BlockSpec((G, H), lambda i: (0, 0)),
            pl.BlockSpec((2 * H, H), lambda i: (0, 0)),
            pl.BlockSpec((1, H), lambda i: (0, 0)),
            pl.BlockSpec((H, OUT), lambda i: (0, 0)),
            pl.BlockSpec((1, OUT), lambda i: (0, 0)),
        ],
        out_specs=pl.BlockSpec((G, OUT), lambda i: (0, 0)),
        out_shape=jax.ShapeDtypeStruct((G, OUT), _f32),
        scratch_shapes=[pltpu.VMEM((G, H), _f32)],
    )(y, dinv, aggp, b, bat2, news, lin1_W, lin1_b, lin2_W, lin2_b)


@jax.jit
def kernel(x, edge_index, batch, W, b, lin0_W, lin0_b, lin1_W, lin1_b,
           lin2_W, lin2_b):
    src = edge_index[0]
    dst = edge_index[1]
    batch = batch.astype(_i32)

    degp = _stats_call(dst)
    degT = degp.T

    y, dinv, news = _dense_call(x, degT, W, batch.reshape(1, N), lin0_W,
                                lin0_b.reshape(1, H))

    z = jnp.zeros((N, H), _f32)
    aggp = _scatter_call(src, dst, y, z)

    out = _readout_call(y, dinv, aggp, b.reshape(1, H),
                        batch.reshape(N, 1), news, lin1_W,
                        lin1_b.reshape(1, H), lin2_W,
                        lin2_b.reshape(1, OUT))
    return out
